# Initial kernel scaffold; baseline (speedup 1.0000x reference)
import jax
import jax.numpy as jnp
from jax.experimental import pallas as pl


def kernel(obj, rel, frm, act, W_obj, b_obj, W_rel, b_rel, W_frame, b_frame, W_action, b_action, token_type_table, position_table, ln_gamma, ln_beta):
    raise NotImplementedError("write your pallas kernel here")


# speedup vs baseline: 1.5735x; 1.5735x over previous
import functools

import jax
import jax.numpy as jnp
from jax.experimental import pallas as pl
from jax.experimental.pallas import tpu as pltpu

EPS = 1e-12
TB = 32


def _fused_body(t1, t2, t3,
                obj_ref, rel_ref, frm_ref, act_ref,
                w_ref, bias_ref, pos_ref, gamma_ref, beta_ref, out_ref):
    t = pl.program_id(0)
    sid = ((t >= t1).astype(jnp.int32) + (t >= t2).astype(jnp.int32)
           + (t >= t3).astype(jnp.int32))
    x = jax.lax.switch(sid, [
        lambda: obj_ref[...],
        lambda: rel_ref[...],
        lambda: frm_ref[...],
        lambda: act_ref[...],
    ])
    b, tb, d = x.shape
    h = out_ref.shape[-1]
    rows = x.reshape(b * tb, d).astype(jnp.bfloat16)
    w = w_ref[0].astype(jnp.bfloat16)
    y = jnp.dot(rows, w, preferred_element_type=jnp.float32)
    y = y.reshape(b, tb, h)
    y = y + bias_ref[0, 0][None, None, :] + pos_ref[...][None, :, :]
    mean = jnp.mean(y, axis=-1, keepdims=True)
    yc = y - mean
    var = jnp.mean(yc * yc, axis=-1, keepdims=True)
    inv = jax.lax.rsqrt(var + EPS)
    out_ref[...] = yc * inv * gamma_ref[0][None, None, :] + beta_ref[0][None, None, :]


def kernel(obj, rel, frm, act, W_obj, b_obj, W_rel, b_rel, W_frame, b_frame,
           W_action, b_action, token_type_table, position_table, ln_gamma, ln_beta):
    B, NO, D = obj.shape
    NR, NF, NA = rel.shape[1], frm.shape[1], act.shape[1]
    T = NO + NR + NF + NA
    H = W_obj.shape[1]

    n_obj, n_rel, n_frm, n_act = NO // TB, NR // TB, NF // TB, NA // TB
    t1 = n_obj
    t2 = t1 + n_rel
    t3 = t2 + n_frm
    num_tiles = t3 + n_act

    w_all = jnp.stack([W_obj, W_rel, W_frame, W_action])
    bias_all = jnp.stack([
        b_obj + token_type_table[1],
        b_rel + token_type_table[2],
        b_frame + token_type_table[3],
        b_action + token_type_table[4],
    ]).reshape(4, 1, H)
    gamma2 = ln_gamma.reshape(1, H)
    beta2 = ln_beta.reshape(1, H)

    def sid_of(t):
        return ((t >= t1).astype(jnp.int32) + (t >= t2).astype(jnp.int32)
                + (t >= t3).astype(jnp.int32))

    def clamp(v, hi):
        return jnp.clip(v, 0, hi)

    in_specs = [
        pl.BlockSpec((B, TB, D), lambda t: (0, clamp(t, n_obj - 1), 0)),
        pl.BlockSpec((B, TB, D), lambda t: (0, clamp(t - t1, n_rel - 1), 0)),
        pl.BlockSpec((B, TB, D), lambda t: (0, clamp(t - t2, n_frm - 1), 0)),
        pl.BlockSpec((B, TB, D), lambda t: (0, clamp(t - t3, n_act - 1), 0)),
        pl.BlockSpec((1, D, H), lambda t: (sid_of(t), 0, 0)),
        pl.BlockSpec((1, 1, H), lambda t: (sid_of(t), 0, 0)),
        pl.BlockSpec((TB, H), lambda t: (t, 0)),
        pl.BlockSpec((1, H), lambda t: (0, 0)),
        pl.BlockSpec((1, H), lambda t: (0, 0)),
    ]

    out = pl.pallas_call(
        functools.partial(_fused_body, t1, t2, t3),
        grid=(num_tiles,),
        in_specs=in_specs,
        out_specs=pl.BlockSpec((B, TB, H), lambda t: (0, t, 0)),
        out_shape=jax.ShapeDtypeStruct((B, T, H), jnp.float32),
    )(obj, rel, frm, act, w_all, bias_all, position_table, gamma2, beta2)

    non_pad_mask = jnp.ones((B, T), dtype=bool)
    return out, non_pad_mask


# speedup vs baseline: 2.5813x; 1.6405x over previous
import functools

import jax
import jax.numpy as jnp
from jax.experimental import pallas as pl
from jax.experimental.pallas import tpu as pltpu

EPS = 1e-12


def _proj_ln(x_ref, w_ref, bias_ref, pos_ref, gamma_ref, beta_ref, out_ref):
    b, tb, d = x_ref.shape
    h = out_ref.shape[-1]
    rows = x_ref[...].reshape(b * tb, d).astype(jnp.bfloat16)
    y = jnp.dot(rows, w_ref[...], preferred_element_type=jnp.float32)
    y = y.reshape(b, tb, h)
    y = y + bias_ref[0][None, None, :] + pos_ref[...][None, :, :]
    mean = jnp.mean(y, axis=-1, keepdims=True)
    yc = y - mean
    var = jnp.mean(yc * yc, axis=-1, keepdims=True)
    inv = jax.lax.rsqrt(var + EPS)
    out_ref[...] = yc * inv * gamma_ref[0][None, None, :] + beta_ref[0][None, None, :]


def _proj_ln_acc(acc_ref, *rest):
    del acc_ref
    _proj_ln(*rest)


def _stream_call(acc, x, w, bias2, position_table, gamma2, beta2,
                 tb, tok_off, T):
    B, N, D = x.shape
    H = w.shape[1]
    n_tiles = N // tb
    off = tok_off // tb

    data_specs = [
        pl.BlockSpec((B, tb, D), lambda t: (0, t, 0)),
        pl.BlockSpec((D, H), lambda t: (0, 0)),
        pl.BlockSpec((1, H), lambda t: (0, 0)),
        pl.BlockSpec((tb, H), lambda t: (t + off, 0)),
        pl.BlockSpec((1, H), lambda t: (0, 0)),
        pl.BlockSpec((1, H), lambda t: (0, 0)),
    ]
    out_spec = pl.BlockSpec((B, tb, H), lambda t: (0, t + off, 0))
    out_shape = jax.ShapeDtypeStruct((B, T, H), jnp.float32)

    if acc is None:
        return pl.pallas_call(
            _proj_ln,
            grid=(n_tiles,),
            in_specs=data_specs,
            out_specs=out_spec,
            out_shape=out_shape,
        )(x, w, bias2, position_table, gamma2, beta2)
    return pl.pallas_call(
        _proj_ln_acc,
        grid=(n_tiles,),
        in_specs=[pl.BlockSpec(memory_space=pl.ANY)] + data_specs,
        out_specs=out_spec,
        out_shape=out_shape,
        input_output_aliases={0: 0},
    )(acc, x, w, bias2, position_table, gamma2, beta2)


def kernel(obj, rel, frm, act, W_obj, b_obj, W_rel, b_rel, W_frame, b_frame,
           W_action, b_action, token_type_table, position_table, ln_gamma, ln_beta):
    B, NO, D = obj.shape
    NR, NF, NA = rel.shape[1], frm.shape[1], act.shape[1]
    T = NO + NR + NF + NA
    H = W_obj.shape[1]

    gamma2 = ln_gamma.reshape(1, H)
    beta2 = ln_beta.reshape(1, H)

    def pick_tb(n, off):
        for tb in (128, 64, 32):
            if n % tb == 0 and off % tb == 0:
                return tb
        raise ValueError(f"stream length {n} at offset {off} not tileable")

    streams = [
        (obj, W_obj, b_obj, 1, pick_tb(NO, 0), 0),
        (rel, W_rel, b_rel, 2, pick_tb(NR, NO), NO),
        (frm, W_frame, b_frame, 3, pick_tb(NF, NO + NR), NO + NR),
        (act, W_action, b_action, 4, pick_tb(NA, NO + NR + NF), NO + NR + NF),
    ]
    acc = None
    for x, w, b, tt_row, tb, tok_off in streams:
        w16 = w.astype(jnp.bfloat16)
        bias2 = (b + token_type_table[tt_row]).reshape(1, H)
        acc = _stream_call(acc, x, w16, bias2, position_table, gamma2, beta2,
                           tb, tok_off, T)

    non_pad_mask = jnp.ones((B, T), dtype=bool)
    return acc, non_pad_mask
